# Initial kernel scaffold; baseline (speedup 1.0000x reference)
import jax
import jax.numpy as jnp
from jax.experimental import pallas as pl


def kernel(x1, edge_index1, batch1, x2, edge_index2, batch2, params):
    raise NotImplementedError("write your pallas kernel here")


# speedup vs baseline: 15.4665x; 15.4665x over previous
import functools

import jax
import jax.numpy as jnp
from jax import lax
from jax.experimental import pallas as pl
from jax.experimental.pallas import tpu as pltpu
from jax.experimental.pallas import tpu_sc as plsc

N = 10000
NPAD = 10240
D = 128
B = 64
E = 320000
TPV = 16
CHUNK = 128
EPT = 20480
NCHUNK = EPT // CHUNK
EPV = EPT * TPV
QCHUNK = 40
RPT = NPAD // TPV
RB = 512
GI = NPAD // RB


def _deg_body(dst_hbm, out_hbm, didx, ones_v, zb, acc):
    c = lax.axis_index("c")
    s = lax.axis_index("s")
    tid = c * TPV + s
    for k in range(CHUNK // 16):
        ones_v[pl.ds(k * 16, 16)] = jnp.ones((16,), jnp.float32)
    for k in range(RPT // 16):
        zb[pl.ds(k * 16, 16)] = jnp.zeros((16,), jnp.float32)
    pltpu.sync_copy(zb, acc.at[pl.ds(s * RPT, RPT)])
    plsc.subcore_barrier()
    pltpu.sync_copy(dst_hbm.at[tid], didx)

    def _count(j, carry):
        pltpu.sync_copy(ones_v, acc.at[didx.at[j]], add=True)
        return carry

    lax.fori_loop(0, NCHUNK, _count, 0)
    plsc.subcore_barrier()
    pltpu.sync_copy(acc.at[pl.ds(s * RPT, RPT)], out_hbm.at[c, pl.ds(s * RPT, RPT)])


@functools.cache
def _deg_call():
    return pl.kernel(
        _deg_body,
        out_type=jax.ShapeDtypeStruct((2, NPAD), jnp.float32),
        mesh=plsc.VectorSubcoreMesh(core_axis_name="c", subcore_axis_name="s"),
        scratch_types=[
            pltpu.VMEM((NCHUNK, CHUNK), jnp.int32),
            pltpu.VMEM((CHUNK,), jnp.float32),
            pltpu.VMEM((RPT,), jnp.float32),
            pltpu.VMEM_SHARED((NPAD,), jnp.float32),
        ],
    )


def _edge_body(src_hbm, dst_hbm, gp_hbm, out_hbm, sidx, didx, rows, acc):
    c = lax.axis_index("c")
    s = lax.axis_index("s")
    tid = c * TPV + s

    def _zrow(r, carry):
        for k in range(D // 16):
            rows[r, pl.ds(k * 16, 16)] = jnp.zeros((16,), jnp.float32)
        return carry

    lax.fori_loop(0, CHUNK, _zrow, 0)
    for k in range(RPT // CHUNK):
        pltpu.sync_copy(rows, acc.at[pl.ds(s * RPT + k * CHUNK, CHUNK)])
    plsc.subcore_barrier()

    def _chunk(j, carry):
        pltpu.sync_copy(gp_hbm.at[sidx.at[j]], rows)
        pltpu.sync_copy(rows, acc.at[didx.at[j]], add=True)
        return carry

    for q in range(NCHUNK // QCHUNK):
        pltpu.sync_copy(src_hbm.at[tid, pl.ds(q * QCHUNK, QCHUNK)], sidx)
        pltpu.sync_copy(dst_hbm.at[tid, pl.ds(q * QCHUNK, QCHUNK)], didx)
        lax.fori_loop(0, QCHUNK, _chunk, 0)
    plsc.subcore_barrier()
    pltpu.sync_copy(acc.at[pl.ds(s * RPT, RPT)], out_hbm.at[c, pl.ds(s * RPT, RPT)])


@functools.cache
def _edge_call():
    return pl.kernel(
        _edge_body,
        out_type=jax.ShapeDtypeStruct((2, NPAD, D), jnp.float32),
        mesh=plsc.VectorSubcoreMesh(core_axis_name="c", subcore_axis_name="s"),
        scratch_types=[
            pltpu.VMEM((QCHUNK, CHUNK), jnp.int32),
            pltpu.VMEM((QCHUNK, CHUNK), jnp.int32),
            pltpu.VMEM((CHUNK, D), jnp.float32),
            pltpu.VMEM_SHARED((NPAD, D), jnp.float32),
        ],
    )


def _l1_body(x_ref, deg_ref, w_ref, gp_ref, dinv_ref):
    dinv = lax.rsqrt(deg_ref[0] + 1.0)
    dinv_ref[0] = dinv
    g = jnp.dot(x_ref[0], w_ref[...], preferred_element_type=jnp.float32)
    gp_ref[0] = g * dinv


def _l2_body(t_ref, gp_ref, dinv_ref, b_ref, w_ref, u_ref, gpn_ref):
    dinv = dinv_ref[0]
    u = jnp.maximum((t_ref[0] + gp_ref[0]) * dinv + b_ref[...], 0.0)
    u_ref[0] = u
    gpn_ref[0] = jnp.dot(u, w_ref[...], preferred_element_type=jnp.float32) * dinv


def _l3_body(t_ref, gp_ref, up_ref, dinv_ref, b_ref, w_ref, u_ref, gpn_ref):
    dinv = dinv_ref[0]
    u = jnp.maximum((t_ref[0] + gp_ref[0]) * dinv + b_ref[...], 0.0) + up_ref[0]
    u_ref[0] = u
    gpn_ref[0] = jnp.dot(u, w_ref[...], preferred_element_type=jnp.float32) * dinv


def _pool_body(t_ref, gp_ref, up_ref, dinv_ref, b_ref, batch_ref,
               sums_ref, cnt_ref):
    i = pl.program_id(1)
    u4 = jnp.maximum((t_ref[0] + gp_ref[0]) * dinv_ref[0] + b_ref[...], 0.0) \
        + up_ref[0]
    bvec = batch_ref[0]
    rows = lax.broadcasted_iota(jnp.int32, (B, RB), 0)
    oh = (bvec == rows).astype(jnp.float32)
    psum = jnp.dot(oh, u4, preferred_element_type=jnp.float32,
                   precision=lax.Precision.HIGHEST)
    pcnt = jnp.sum(oh, axis=1, keepdims=True)

    @pl.when(i == 0)
    def _():
        sums_ref[0] = psum
        cnt_ref[0] = pcnt

    @pl.when(i > 0)
    def _():
        sums_ref[0] += psum
        cnt_ref[0] += pcnt


def _head_body(sums_ref, cnt_ref, pw1, pb1, g1, be1, pw2, pb2, g2, be2,
               qw1, qb1, qw2, qb2, p_ref, z_ref):
    h = sums_ref[0] / jnp.maximum(cnt_ref[0], 1.0)
    z = jnp.dot(h, pw1[...], preferred_element_type=jnp.float32) + pb1[...]
    mu = jnp.mean(z, axis=0, keepdims=True)
    var = jnp.mean((z - mu) ** 2, axis=0, keepdims=True)
    z = jnp.maximum(g1[...] * (z - mu) * lax.rsqrt(var + 1e-5) + be1[...], 0.0)
    z = jnp.dot(z, pw2[...], preferred_element_type=jnp.float32) + pb2[...]
    mu = jnp.mean(z, axis=0, keepdims=True)
    var = jnp.mean((z - mu) ** 2, axis=0, keepdims=True)
    z = g2[...] * (z - mu) * lax.rsqrt(var + 1e-5) + be2[...]
    z_ref[0] = z
    q = jnp.maximum(
        jnp.dot(z, qw1[...], preferred_element_type=jnp.float32) + qb1[...], 0.0)
    p_ref[0] = jnp.dot(q, qw2[...], preferred_element_type=jnp.float32) + qb2[...]


def _row_spec(last=D):
    return pl.BlockSpec((1, RB, last), lambda v, i: (v, i, 0))


def _full_spec(shape):
    return pl.BlockSpec(shape, lambda v, i: tuple(0 for _ in shape))


def _tc_layer1(x_s, deg, w1):
    return pl.pallas_call(
        _l1_body,
        grid=(2, GI),
        in_specs=[_row_spec(), _row_spec(1), _full_spec((D, D))],
        out_specs=[_row_spec(), _row_spec(1)],
        out_shape=[jax.ShapeDtypeStruct((2, NPAD, D), jnp.float32),
                   jax.ShapeDtypeStruct((2, NPAD, 1), jnp.float32)],
    )(x_s, deg, w1)


def _tc_layer2(t, gp, dinv, b, w):
    return pl.pallas_call(
        _l2_body,
        grid=(2, GI),
        in_specs=[_row_spec(), _row_spec(), _row_spec(1),
                  _full_spec((1, D)), _full_spec((D, D))],
        out_specs=[_row_spec(), _row_spec()],
        out_shape=[jax.ShapeDtypeStruct((2, NPAD, D), jnp.float32),
                   jax.ShapeDtypeStruct((2, NPAD, D), jnp.float32)],
    )(t, gp, dinv, b, w)


def _tc_layer3(t, gp, up, dinv, b, w):
    return pl.pallas_call(
        _l3_body,
        grid=(2, GI),
        in_specs=[_row_spec(), _row_spec(), _row_spec(), _row_spec(1),
                  _full_spec((1, D)), _full_spec((D, D))],
        out_specs=[_row_spec(), _row_spec()],
        out_shape=[jax.ShapeDtypeStruct((2, NPAD, D), jnp.float32),
                   jax.ShapeDtypeStruct((2, NPAD, D), jnp.float32)],
    )(t, gp, up, dinv, b, w)


def _tc_pool(t, gp, up, dinv, b, batch_s):
    return pl.pallas_call(
        _pool_body,
        grid=(2, GI),
        in_specs=[_row_spec(), _row_spec(), _row_spec(), _row_spec(1),
                  _full_spec((1, D)),
                  pl.BlockSpec((1, 1, RB), lambda v, i: (v, 0, i))],
        out_specs=[pl.BlockSpec((1, B, D), lambda v, i: (v, 0, 0)),
                   pl.BlockSpec((1, B, 1), lambda v, i: (v, 0, 0))],
        out_shape=[jax.ShapeDtypeStruct((2, B, D), jnp.float32),
                   jax.ShapeDtypeStruct((2, B, 1), jnp.float32)],
    )(t, gp, up, dinv, b, batch_s)


def _tc_head(sums, cnt, p):
    specs = [pl.BlockSpec((1, B, D), lambda v: (v, 0, 0)),
             pl.BlockSpec((1, B, 1), lambda v: (v, 0, 0))]
    args = [sums, cnt]
    for w, bias in ((p['pW1'], p['pb1']), (p['g1'], p['be1']),
                    (p['pW2'], p['pb2']), (p['g2'], p['be2']),
                    (p['qW1'], p['qb1']), (p['qW2'], p['qb2'])):
        for a in (w, bias):
            a2 = a.reshape((1, -1)) if a.ndim == 1 else a
            args.append(a2)
            specs.append(pl.BlockSpec(a2.shape, lambda v: (0, 0)))
    return pl.pallas_call(
        _head_body,
        grid=(2,),
        in_specs=specs,
        out_specs=[pl.BlockSpec((1, B, D), lambda v: (v, 0, 0)),
                   pl.BlockSpec((1, B, D), lambda v: (v, 0, 0))],
        out_shape=[jax.ShapeDtypeStruct((2, B, D), jnp.float32),
                   jax.ShapeDtypeStruct((2, B, D), jnp.float32)],
    )(*args)


def _prep_edges(ei, view):
    src = ei[0].astype(jnp.int32)
    dst = ei[1].astype(jnp.int32)
    pad = EPV - E
    ar = jnp.arange(pad, dtype=jnp.int32)
    src = jnp.concatenate([src, ar % N]) + view * NPAD
    dst = jnp.concatenate([dst, N + (ar % (NPAD - N))])
    return src, dst


def kernel(x1, edge_index1, batch1, x2, edge_index2, batch2, params):
    p = params
    x_s = jnp.pad(jnp.stack([x1, x2]).astype(jnp.float32),
                  ((0, 0), (0, NPAD - N), (0, 0)))
    s1, d1 = _prep_edges(edge_index1, 0)
    s2, d2 = _prep_edges(edge_index2, 1)
    e_src = jnp.stack([s1, s2]).reshape(2 * TPV, NCHUNK, CHUNK)
    e_dst = jnp.stack([d1, d2]).reshape(2 * TPV, NCHUNK, CHUNK)
    batch_s = jnp.pad(jnp.stack([batch1, batch2]).astype(jnp.int32),
                      ((0, 0), (0, NPAD - N)),
                      constant_values=B).reshape(2, 1, NPAD)

    deg = _deg_call()(e_dst).reshape(2, NPAD, 1)
    gp1, dinv = _tc_layer1(x_s, deg, p['W1'])
    t1 = _edge_call()(e_src, e_dst, gp1.reshape(2 * NPAD, D))
    u2, gp2 = _tc_layer2(t1, gp1, dinv, p['b1'].reshape(1, D), p['W2'])
    t2 = _edge_call()(e_src, e_dst, gp2.reshape(2 * NPAD, D))
    u3, gp3 = _tc_layer3(t2, gp2, u2, dinv, p['b2'].reshape(1, D), p['W3'])
    t3 = _edge_call()(e_src, e_dst, gp3.reshape(2 * NPAD, D))
    sums, cnt = _tc_pool(t3, gp3, u3, dinv, p['b3'].reshape(1, D), batch_s)
    p_out, z_out = _tc_head(sums, cnt, p)
    return (p_out[0], p_out[1], z_out[0], z_out[1])


# speedup vs baseline: 17.9201x; 1.1586x over previous
import functools

import jax
import jax.numpy as jnp
from jax import lax
from jax.experimental import pallas as pl
from jax.experimental.pallas import tpu as pltpu
from jax.experimental.pallas import tpu_sc as plsc

N = 10000
NPAD = 10240
D = 128
B = 64
E = 320000
TPV = 16
CHUNK = 128
EPT = 20480
NCHUNK = EPT // CHUNK
EPV = EPT * TPV
QCHUNK = 40
RPT = NPAD // TPV
RB = 512
GI = NPAD // RB


def _deg_body(dst_hbm, out_hbm, didx, ones_v, zb, acc):
    c = lax.axis_index("c")
    s = lax.axis_index("s")
    tid = c * TPV + s
    for k in range(CHUNK // 16):
        ones_v[pl.ds(k * 16, 16)] = jnp.ones((16,), jnp.float32)
    for k in range(RPT // 16):
        zb[pl.ds(k * 16, 16)] = jnp.zeros((16,), jnp.float32)
    pltpu.sync_copy(zb, acc.at[pl.ds(s * RPT, RPT)])
    plsc.subcore_barrier()
    pltpu.sync_copy(dst_hbm.at[tid], didx)

    def _count(j, carry):
        pltpu.sync_copy(ones_v, acc.at[didx.at[j]], add=True)
        return carry

    lax.fori_loop(0, NCHUNK, _count, 0)
    plsc.subcore_barrier()
    pltpu.sync_copy(acc.at[pl.ds(s * RPT, RPT)], out_hbm.at[c, pl.ds(s * RPT, RPT)])


@functools.cache
def _deg_call():
    return pl.kernel(
        _deg_body,
        out_type=jax.ShapeDtypeStruct((2, NPAD), jnp.float32),
        mesh=plsc.VectorSubcoreMesh(core_axis_name="c", subcore_axis_name="s"),
        scratch_types=[
            pltpu.VMEM((NCHUNK, CHUNK), jnp.int32),
            pltpu.VMEM((CHUNK,), jnp.float32),
            pltpu.VMEM((RPT,), jnp.float32),
            pltpu.VMEM_SHARED((NPAD,), jnp.float32),
        ],
    )


def _edge_body(src_hbm, dst_hbm, gp_hbm, out_hbm, sidx, didx, rows0, rows1,
               acc, sg0, sg1, ss0, ss1):
    c = lax.axis_index("c")
    s = lax.axis_index("s")
    tid = c * TPV + s

    def _zrow(r, carry):
        for k in range(D // 16):
            rows0[r, pl.ds(k * 16, 16)] = jnp.zeros((16,), jnp.float32)
        return carry

    lax.fori_loop(0, CHUNK, _zrow, 0)
    for k in range(RPT // CHUNK):
        pltpu.sync_copy(rows0, acc.at[pl.ds(s * RPT + k * CHUNK, CHUNK)])
    plsc.subcore_barrier()

    def _chunk(m, carry):
        j0 = 2 * m
        j1 = j0 + 1
        g0 = pltpu.async_copy(gp_hbm.at[sidx.at[j0]], rows0, sg0)
        g1 = pltpu.async_copy(gp_hbm.at[sidx.at[j1]], rows1, sg1)
        g0.wait()
        s0 = pltpu.async_copy(rows0, acc.at[didx.at[j0]], ss0, add=True)
        g1.wait()
        s1 = pltpu.async_copy(rows1, acc.at[didx.at[j1]], ss1, add=True)
        s0.wait()
        s1.wait()
        return carry

    for q in range(NCHUNK // QCHUNK):
        pltpu.sync_copy(src_hbm.at[tid, pl.ds(q * QCHUNK, QCHUNK)], sidx)
        pltpu.sync_copy(dst_hbm.at[tid, pl.ds(q * QCHUNK, QCHUNK)], didx)
        lax.fori_loop(0, QCHUNK // 2, _chunk, 0)
    plsc.subcore_barrier()
    pltpu.sync_copy(acc.at[pl.ds(s * RPT, RPT)], out_hbm.at[c, pl.ds(s * RPT, RPT)])


@functools.cache
def _edge_call():
    return pl.kernel(
        _edge_body,
        out_type=jax.ShapeDtypeStruct((2, NPAD, D), jnp.float32),
        mesh=plsc.VectorSubcoreMesh(core_axis_name="c", subcore_axis_name="s"),
        scratch_types=[
            pltpu.VMEM((QCHUNK, CHUNK), jnp.int32),
            pltpu.VMEM((QCHUNK, CHUNK), jnp.int32),
            pltpu.VMEM((CHUNK, D), jnp.float32),
            pltpu.VMEM((CHUNK, D), jnp.float32),
            pltpu.VMEM_SHARED((NPAD, D), jnp.float32),
            pltpu.SemaphoreType.DMA,
            pltpu.SemaphoreType.DMA,
            pltpu.SemaphoreType.DMA,
            pltpu.SemaphoreType.DMA,
        ],
    )


def _l1_body(x_ref, deg_ref, w_ref, gp_ref, dinv_ref):
    dinv = lax.rsqrt(deg_ref[0] + 1.0)
    dinv_ref[0] = dinv
    g = jnp.dot(x_ref[0], w_ref[...], preferred_element_type=jnp.float32)
    gp_ref[0] = g * dinv


def _l2_body(t_ref, gp_ref, dinv_ref, b_ref, w_ref, u_ref, gpn_ref):
    dinv = dinv_ref[0]
    u = jnp.maximum((t_ref[0] + gp_ref[0]) * dinv + b_ref[...], 0.0)
    u_ref[0] = u
    gpn_ref[0] = jnp.dot(u, w_ref[...], preferred_element_type=jnp.float32) * dinv


def _l3_body(t_ref, gp_ref, up_ref, dinv_ref, b_ref, w_ref, u_ref, gpn_ref):
    dinv = dinv_ref[0]
    u = jnp.maximum((t_ref[0] + gp_ref[0]) * dinv + b_ref[...], 0.0) + up_ref[0]
    u_ref[0] = u
    gpn_ref[0] = jnp.dot(u, w_ref[...], preferred_element_type=jnp.float32) * dinv


def _pool_body(t_ref, gp_ref, up_ref, dinv_ref, b_ref, batch_ref,
               sums_ref, cnt_ref):
    i = pl.program_id(1)
    u4 = jnp.maximum((t_ref[0] + gp_ref[0]) * dinv_ref[0] + b_ref[...], 0.0) \
        + up_ref[0]
    bvec = batch_ref[0]
    rows = lax.broadcasted_iota(jnp.int32, (B, RB), 0)
    oh = (bvec == rows).astype(jnp.float32)
    psum = jnp.dot(oh, u4, preferred_element_type=jnp.float32,
                   precision=lax.Precision.HIGHEST)
    pcnt = jnp.sum(oh, axis=1, keepdims=True)

    @pl.when(i == 0)
    def _():
        sums_ref[0] = psum
        cnt_ref[0] = pcnt

    @pl.when(i > 0)
    def _():
        sums_ref[0] += psum
        cnt_ref[0] += pcnt


def _head_body(sums_ref, cnt_ref, pw1, pb1, g1, be1, pw2, pb2, g2, be2,
               qw1, qb1, qw2, qb2, p_ref, z_ref):
    h = sums_ref[0] / jnp.maximum(cnt_ref[0], 1.0)
    z = jnp.dot(h, pw1[...], preferred_element_type=jnp.float32) + pb1[...]
    mu = jnp.mean(z, axis=0, keepdims=True)
    var = jnp.mean((z - mu) ** 2, axis=0, keepdims=True)
    z = jnp.maximum(g1[...] * (z - mu) * lax.rsqrt(var + 1e-5) + be1[...], 0.0)
    z = jnp.dot(z, pw2[...], preferred_element_type=jnp.float32) + pb2[...]
    mu = jnp.mean(z, axis=0, keepdims=True)
    var = jnp.mean((z - mu) ** 2, axis=0, keepdims=True)
    z = g2[...] * (z - mu) * lax.rsqrt(var + 1e-5) + be2[...]
    z_ref[0] = z
    q = jnp.maximum(
        jnp.dot(z, qw1[...], preferred_element_type=jnp.float32) + qb1[...], 0.0)
    p_ref[0] = jnp.dot(q, qw2[...], preferred_element_type=jnp.float32) + qb2[...]


def _row_spec(last=D):
    return pl.BlockSpec((1, RB, last), lambda v, i: (v, i, 0))


def _full_spec(shape):
    return pl.BlockSpec(shape, lambda v, i: tuple(0 for _ in shape))


def _tc_layer1(x_s, deg, w1):
    return pl.pallas_call(
        _l1_body,
        grid=(2, GI),
        in_specs=[_row_spec(), _row_spec(1), _full_spec((D, D))],
        out_specs=[_row_spec(), _row_spec(1)],
        out_shape=[jax.ShapeDtypeStruct((2, NPAD, D), jnp.float32),
                   jax.ShapeDtypeStruct((2, NPAD, 1), jnp.float32)],
    )(x_s, deg, w1)


def _tc_layer2(t, gp, dinv, b, w):
    return pl.pallas_call(
        _l2_body,
        grid=(2, GI),
        in_specs=[_row_spec(), _row_spec(), _row_spec(1),
                  _full_spec((1, D)), _full_spec((D, D))],
        out_specs=[_row_spec(), _row_spec()],
        out_shape=[jax.ShapeDtypeStruct((2, NPAD, D), jnp.float32),
                   jax.ShapeDtypeStruct((2, NPAD, D), jnp.float32)],
    )(t, gp, dinv, b, w)


def _tc_layer3(t, gp, up, dinv, b, w):
    return pl.pallas_call(
        _l3_body,
        grid=(2, GI),
        in_specs=[_row_spec(), _row_spec(), _row_spec(), _row_spec(1),
                  _full_spec((1, D)), _full_spec((D, D))],
        out_specs=[_row_spec(), _row_spec()],
        out_shape=[jax.ShapeDtypeStruct((2, NPAD, D), jnp.float32),
                   jax.ShapeDtypeStruct((2, NPAD, D), jnp.float32)],
    )(t, gp, up, dinv, b, w)


def _tc_pool(t, gp, up, dinv, b, batch_s):
    return pl.pallas_call(
        _pool_body,
        grid=(2, GI),
        in_specs=[_row_spec(), _row_spec(), _row_spec(), _row_spec(1),
                  _full_spec((1, D)),
                  pl.BlockSpec((1, 1, RB), lambda v, i: (v, 0, i))],
        out_specs=[pl.BlockSpec((1, B, D), lambda v, i: (v, 0, 0)),
                   pl.BlockSpec((1, B, 1), lambda v, i: (v, 0, 0))],
        out_shape=[jax.ShapeDtypeStruct((2, B, D), jnp.float32),
                   jax.ShapeDtypeStruct((2, B, 1), jnp.float32)],
    )(t, gp, up, dinv, b, batch_s)


def _tc_head(sums, cnt, p):
    specs = [pl.BlockSpec((1, B, D), lambda v: (v, 0, 0)),
             pl.BlockSpec((1, B, 1), lambda v: (v, 0, 0))]
    args = [sums, cnt]
    for w, bias in ((p['pW1'], p['pb1']), (p['g1'], p['be1']),
                    (p['pW2'], p['pb2']), (p['g2'], p['be2']),
                    (p['qW1'], p['qb1']), (p['qW2'], p['qb2'])):
        for a in (w, bias):
            a2 = a.reshape((1, -1)) if a.ndim == 1 else a
            args.append(a2)
            specs.append(pl.BlockSpec(a2.shape, lambda v: (0, 0)))
    return pl.pallas_call(
        _head_body,
        grid=(2,),
        in_specs=specs,
        out_specs=[pl.BlockSpec((1, B, D), lambda v: (v, 0, 0)),
                   pl.BlockSpec((1, B, D), lambda v: (v, 0, 0))],
        out_shape=[jax.ShapeDtypeStruct((2, B, D), jnp.float32),
                   jax.ShapeDtypeStruct((2, B, D), jnp.float32)],
    )(*args)


def _prep_edges(ei, view):
    src = ei[0].astype(jnp.int32)
    dst = ei[1].astype(jnp.int32)
    pad = EPV - E
    ar = jnp.arange(pad, dtype=jnp.int32)
    src = jnp.concatenate([src, ar % N]) + view * NPAD
    dst = jnp.concatenate([dst, N + (ar % (NPAD - N))])
    return src, dst


def kernel(x1, edge_index1, batch1, x2, edge_index2, batch2, params):
    p = params
    x_s = jnp.pad(jnp.stack([x1, x2]).astype(jnp.float32),
                  ((0, 0), (0, NPAD - N), (0, 0)))
    s1, d1 = _prep_edges(edge_index1, 0)
    s2, d2 = _prep_edges(edge_index2, 1)
    e_src = jnp.stack([s1, s2]).reshape(2 * TPV, NCHUNK, CHUNK)
    e_dst = jnp.stack([d1, d2]).reshape(2 * TPV, NCHUNK, CHUNK)
    batch_s = jnp.pad(jnp.stack([batch1, batch2]).astype(jnp.int32),
                      ((0, 0), (0, NPAD - N)),
                      constant_values=B).reshape(2, 1, NPAD)

    deg = _deg_call()(e_dst).reshape(2, NPAD, 1)
    gp1, dinv = _tc_layer1(x_s, deg, p['W1'])
    t1 = _edge_call()(e_src, e_dst, gp1.reshape(2 * NPAD, D))
    u2, gp2 = _tc_layer2(t1, gp1, dinv, p['b1'].reshape(1, D), p['W2'])
    t2 = _edge_call()(e_src, e_dst, gp2.reshape(2 * NPAD, D))
    u3, gp3 = _tc_layer3(t2, gp2, u2, dinv, p['b2'].reshape(1, D), p['W3'])
    t3 = _edge_call()(e_src, e_dst, gp3.reshape(2 * NPAD, D))
    sums, cnt = _tc_pool(t3, gp3, u3, dinv, p['b3'].reshape(1, D), batch_s)
    p_out, z_out = _tc_head(sums, cnt, p)
    return (p_out[0], p_out[1], z_out[0], z_out[1])


# speedup vs baseline: 18.2265x; 1.0171x over previous
import functools

import jax
import jax.numpy as jnp
from jax import lax
from jax.experimental import pallas as pl
from jax.experimental.pallas import tpu as pltpu
from jax.experimental.pallas import tpu_sc as plsc

N = 10000
NPAD = 10240
D = 128
B = 64
E = 320000
TPV = 16
CHUNK = 128
EPT = 20480
NCHUNK = EPT // CHUNK
EPV = EPT * TPV
QCHUNK = 40
RPT = NPAD // TPV
RB = 512
GI = NPAD // RB


def _deg_body(dst_hbm, out_hbm, didx, ones_v, zb, acc):
    c = lax.axis_index("c")
    s = lax.axis_index("s")
    tid = c * TPV + s
    for k in range(CHUNK // 16):
        ones_v[pl.ds(k * 16, 16)] = jnp.ones((16,), jnp.float32)
    for k in range(RPT // 16):
        zb[pl.ds(k * 16, 16)] = jnp.zeros((16,), jnp.float32)
    pltpu.sync_copy(zb, acc.at[pl.ds(s * RPT, RPT)])
    plsc.subcore_barrier()
    pltpu.sync_copy(dst_hbm.at[tid], didx)

    def _count(j, carry):
        pltpu.sync_copy(ones_v, acc.at[didx.at[j]], add=True)
        return carry

    lax.fori_loop(0, NCHUNK, _count, 0)
    plsc.subcore_barrier()
    pltpu.sync_copy(acc.at[pl.ds(s * RPT, RPT)], out_hbm.at[c, pl.ds(s * RPT, RPT)])


@functools.cache
def _deg_call():
    return pl.kernel(
        _deg_body,
        out_type=jax.ShapeDtypeStruct((2, NPAD), jnp.float32),
        mesh=plsc.VectorSubcoreMesh(core_axis_name="c", subcore_axis_name="s"),
        scratch_types=[
            pltpu.VMEM((NCHUNK, CHUNK), jnp.int32),
            pltpu.VMEM((CHUNK,), jnp.float32),
            pltpu.VMEM((RPT,), jnp.float32),
            pltpu.VMEM_SHARED((NPAD,), jnp.float32),
        ],
    )


def _edge_body(src_hbm, dst_hbm, gp_hbm, out_hbm, sidx, didx, rows0, rows1,
               acc, sg0, sg1, ss0, ss1):
    c = lax.axis_index("c")
    s = lax.axis_index("s")
    tid = c * TPV + s

    def _zrow(r, carry):
        for k in range(D // 16):
            rows0[r, pl.ds(k * 16, 16)] = jnp.zeros((16,), jnp.float32)
        return carry

    lax.fori_loop(0, CHUNK, _zrow, 0)
    for k in range(RPT // CHUNK):
        pltpu.sync_copy(rows0, acc.at[pl.ds(s * RPT + k * CHUNK, CHUNK)])
    plsc.subcore_barrier()

    def _chunk(m, carry):
        j0 = 2 * m
        j1 = j0 + 1
        pltpu.make_async_copy(gp_hbm.at[sidx.at[j0]], rows0, sg0).wait()
        pltpu.async_copy(rows0, acc.at[didx.at[j0]], ss0, add=True)
        pltpu.make_async_copy(gp_hbm.at[sidx.at[j1]], rows1, sg1).wait()
        pltpu.async_copy(rows1, acc.at[didx.at[j1]], ss1, add=True)
        pltpu.make_async_copy(rows0, acc.at[didx.at[j0]], ss0).wait()

        @pl.when(j0 + 2 < QCHUNK)
        def _():
            pltpu.async_copy(gp_hbm.at[sidx.at[j0 + 2]], rows0, sg0)

        pltpu.make_async_copy(rows1, acc.at[didx.at[j1]], ss1).wait()

        @pl.when(j1 + 2 < QCHUNK)
        def _():
            pltpu.async_copy(gp_hbm.at[sidx.at[j1 + 2]], rows1, sg1)

        return carry

    for q in range(NCHUNK // QCHUNK):
        pltpu.sync_copy(src_hbm.at[tid, pl.ds(q * QCHUNK, QCHUNK)], sidx)
        pltpu.sync_copy(dst_hbm.at[tid, pl.ds(q * QCHUNK, QCHUNK)], didx)
        pltpu.async_copy(gp_hbm.at[sidx.at[0]], rows0, sg0)
        pltpu.async_copy(gp_hbm.at[sidx.at[1]], rows1, sg1)
        lax.fori_loop(0, QCHUNK // 2, _chunk, 0)
    plsc.subcore_barrier()
    pltpu.sync_copy(acc.at[pl.ds(s * RPT, RPT)], out_hbm.at[c, pl.ds(s * RPT, RPT)])


@functools.cache
def _edge_call():
    return pl.kernel(
        _edge_body,
        out_type=jax.ShapeDtypeStruct((2, NPAD, D), jnp.float32),
        mesh=plsc.VectorSubcoreMesh(core_axis_name="c", subcore_axis_name="s"),
        scratch_types=[
            pltpu.VMEM((QCHUNK, CHUNK), jnp.int32),
            pltpu.VMEM((QCHUNK, CHUNK), jnp.int32),
            pltpu.VMEM((CHUNK, D), jnp.float32),
            pltpu.VMEM((CHUNK, D), jnp.float32),
            pltpu.VMEM_SHARED((NPAD, D), jnp.float32),
            pltpu.SemaphoreType.DMA,
            pltpu.SemaphoreType.DMA,
            pltpu.SemaphoreType.DMA,
            pltpu.SemaphoreType.DMA,
        ],
    )


def _l1_body(x_ref, deg_ref, w_ref, gp_ref, dinv_ref):
    dinv = lax.rsqrt(deg_ref[0] + 1.0)
    dinv_ref[0] = dinv
    g = jnp.dot(x_ref[0], w_ref[...], preferred_element_type=jnp.float32)
    gp_ref[0] = g * dinv


def _l2_body(t_ref, gp_ref, dinv_ref, b_ref, w_ref, u_ref, gpn_ref):
    dinv = dinv_ref[0]
    u = jnp.maximum((t_ref[0] + gp_ref[0]) * dinv + b_ref[...], 0.0)
    u_ref[0] = u
    gpn_ref[0] = jnp.dot(u, w_ref[...], preferred_element_type=jnp.float32) * dinv


def _l3_body(t_ref, gp_ref, up_ref, dinv_ref, b_ref, w_ref, u_ref, gpn_ref):
    dinv = dinv_ref[0]
    u = jnp.maximum((t_ref[0] + gp_ref[0]) * dinv + b_ref[...], 0.0) + up_ref[0]
    u_ref[0] = u
    gpn_ref[0] = jnp.dot(u, w_ref[...], preferred_element_type=jnp.float32) * dinv


def _pool_body(t_ref, gp_ref, up_ref, dinv_ref, b_ref, batch_ref,
               sums_ref, cnt_ref):
    i = pl.program_id(1)
    u4 = jnp.maximum((t_ref[0] + gp_ref[0]) * dinv_ref[0] + b_ref[...], 0.0) \
        + up_ref[0]
    bvec = batch_ref[0]
    rows = lax.broadcasted_iota(jnp.int32, (B, RB), 0)
    oh = (bvec == rows).astype(jnp.float32)
    psum = jnp.dot(oh, u4, preferred_element_type=jnp.float32,
                   precision=lax.Precision.HIGHEST)
    pcnt = jnp.sum(oh, axis=1, keepdims=True)

    @pl.when(i == 0)
    def _():
        sums_ref[0] = psum
        cnt_ref[0] = pcnt

    @pl.when(i > 0)
    def _():
        sums_ref[0] += psum
        cnt_ref[0] += pcnt


def _head_body(sums_ref, cnt_ref, pw1, pb1, g1, be1, pw2, pb2, g2, be2,
               qw1, qb1, qw2, qb2, p_ref, z_ref):
    h = sums_ref[0] / jnp.maximum(cnt_ref[0], 1.0)
    z = jnp.dot(h, pw1[...], preferred_element_type=jnp.float32) + pb1[...]
    mu = jnp.mean(z, axis=0, keepdims=True)
    var = jnp.mean((z - mu) ** 2, axis=0, keepdims=True)
    z = jnp.maximum(g1[...] * (z - mu) * lax.rsqrt(var + 1e-5) + be1[...], 0.0)
    z = jnp.dot(z, pw2[...], preferred_element_type=jnp.float32) + pb2[...]
    mu = jnp.mean(z, axis=0, keepdims=True)
    var = jnp.mean((z - mu) ** 2, axis=0, keepdims=True)
    z = g2[...] * (z - mu) * lax.rsqrt(var + 1e-5) + be2[...]
    z_ref[0] = z
    q = jnp.maximum(
        jnp.dot(z, qw1[...], preferred_element_type=jnp.float32) + qb1[...], 0.0)
    p_ref[0] = jnp.dot(q, qw2[...], preferred_element_type=jnp.float32) + qb2[...]


def _row_spec(last=D):
    return pl.BlockSpec((1, RB, last), lambda v, i: (v, i, 0))


def _full_spec(shape):
    return pl.BlockSpec(shape, lambda v, i: tuple(0 for _ in shape))


def _tc_layer1(x_s, deg, w1):
    return pl.pallas_call(
        _l1_body,
        grid=(2, GI),
        in_specs=[_row_spec(), _row_spec(1), _full_spec((D, D))],
        out_specs=[_row_spec(), _row_spec(1)],
        out_shape=[jax.ShapeDtypeStruct((2, NPAD, D), jnp.float32),
                   jax.ShapeDtypeStruct((2, NPAD, 1), jnp.float32)],
    )(x_s, deg, w1)


def _tc_layer2(t, gp, dinv, b, w):
    return pl.pallas_call(
        _l2_body,
        grid=(2, GI),
        in_specs=[_row_spec(), _row_spec(), _row_spec(1),
                  _full_spec((1, D)), _full_spec((D, D))],
        out_specs=[_row_spec(), _row_spec()],
        out_shape=[jax.ShapeDtypeStruct((2, NPAD, D), jnp.float32),
                   jax.ShapeDtypeStruct((2, NPAD, D), jnp.float32)],
    )(t, gp, dinv, b, w)


def _tc_layer3(t, gp, up, dinv, b, w):
    return pl.pallas_call(
        _l3_body,
        grid=(2, GI),
        in_specs=[_row_spec(), _row_spec(), _row_spec(), _row_spec(1),
                  _full_spec((1, D)), _full_spec((D, D))],
        out_specs=[_row_spec(), _row_spec()],
        out_shape=[jax.ShapeDtypeStruct((2, NPAD, D), jnp.float32),
                   jax.ShapeDtypeStruct((2, NPAD, D), jnp.float32)],
    )(t, gp, up, dinv, b, w)


def _tc_pool(t, gp, up, dinv, b, batch_s):
    return pl.pallas_call(
        _pool_body,
        grid=(2, GI),
        in_specs=[_row_spec(), _row_spec(), _row_spec(), _row_spec(1),
                  _full_spec((1, D)),
                  pl.BlockSpec((1, 1, RB), lambda v, i: (v, 0, i))],
        out_specs=[pl.BlockSpec((1, B, D), lambda v, i: (v, 0, 0)),
                   pl.BlockSpec((1, B, 1), lambda v, i: (v, 0, 0))],
        out_shape=[jax.ShapeDtypeStruct((2, B, D), jnp.float32),
                   jax.ShapeDtypeStruct((2, B, 1), jnp.float32)],
    )(t, gp, up, dinv, b, batch_s)


def _tc_head(sums, cnt, p):
    specs = [pl.BlockSpec((1, B, D), lambda v: (v, 0, 0)),
             pl.BlockSpec((1, B, 1), lambda v: (v, 0, 0))]
    args = [sums, cnt]
    for w, bias in ((p['pW1'], p['pb1']), (p['g1'], p['be1']),
                    (p['pW2'], p['pb2']), (p['g2'], p['be2']),
                    (p['qW1'], p['qb1']), (p['qW2'], p['qb2'])):
        for a in (w, bias):
            a2 = a.reshape((1, -1)) if a.ndim == 1 else a
            args.append(a2)
            specs.append(pl.BlockSpec(a2.shape, lambda v: (0, 0)))
    return pl.pallas_call(
        _head_body,
        grid=(2,),
        in_specs=specs,
        out_specs=[pl.BlockSpec((1, B, D), lambda v: (v, 0, 0)),
                   pl.BlockSpec((1, B, D), lambda v: (v, 0, 0))],
        out_shape=[jax.ShapeDtypeStruct((2, B, D), jnp.float32),
                   jax.ShapeDtypeStruct((2, B, D), jnp.float32)],
    )(*args)


def _prep_edges(ei, view):
    src = ei[0].astype(jnp.int32)
    dst = ei[1].astype(jnp.int32)
    pad = EPV - E
    ar = jnp.arange(pad, dtype=jnp.int32)
    src = jnp.concatenate([src, ar % N]) + view * NPAD
    dst = jnp.concatenate([dst, N + (ar % (NPAD - N))])
    return src, dst


def kernel(x1, edge_index1, batch1, x2, edge_index2, batch2, params):
    p = params
    x_s = jnp.pad(jnp.stack([x1, x2]).astype(jnp.float32),
                  ((0, 0), (0, NPAD - N), (0, 0)))
    s1, d1 = _prep_edges(edge_index1, 0)
    s2, d2 = _prep_edges(edge_index2, 1)
    e_src = jnp.stack([s1, s2]).reshape(2 * TPV, NCHUNK, CHUNK)
    e_dst = jnp.stack([d1, d2]).reshape(2 * TPV, NCHUNK, CHUNK)
    batch_s = jnp.pad(jnp.stack([batch1, batch2]).astype(jnp.int32),
                      ((0, 0), (0, NPAD - N)),
                      constant_values=B).reshape(2, 1, NPAD)

    deg = _deg_call()(e_dst).reshape(2, NPAD, 1)
    gp1, dinv = _tc_layer1(x_s, deg, p['W1'])
    t1 = _edge_call()(e_src, e_dst, gp1.reshape(2 * NPAD, D))
    u2, gp2 = _tc_layer2(t1, gp1, dinv, p['b1'].reshape(1, D), p['W2'])
    t2 = _edge_call()(e_src, e_dst, gp2.reshape(2 * NPAD, D))
    u3, gp3 = _tc_layer3(t2, gp2, u2, dinv, p['b2'].reshape(1, D), p['W3'])
    t3 = _edge_call()(e_src, e_dst, gp3.reshape(2 * NPAD, D))
    sums, cnt = _tc_pool(t3, gp3, u3, dinv, p['b3'].reshape(1, D), batch_s)
    p_out, z_out = _tc_head(sums, cnt, p)
    return (p_out[0], p_out[1], z_out[0], z_out[1])


# speedup vs baseline: 21.0464x; 1.1547x over previous
import functools

import jax
import jax.numpy as jnp
from jax import lax
from jax.experimental import pallas as pl
from jax.experimental.pallas import tpu as pltpu
from jax.experimental.pallas import tpu_sc as plsc

N = 10000
NPAD = 10240
D = 128
B = 64
E = 320000
TPV = 16
CHUNK = 128
EPT = 20480
NCHUNK = EPT // CHUNK
EPV = EPT * TPV
QCHUNK = 40
ECH = 64
ENCH = EPT // ECH
EQ = 40
NBUF = 4
RPT = NPAD // TPV
RB = 512
GI = NPAD // RB


def _deg_body(dst_hbm, out_hbm, didx, ones_v, zb, acc):
    c = lax.axis_index("c")
    s = lax.axis_index("s")
    tid = c * TPV + s
    for k in range(CHUNK // 16):
        ones_v[pl.ds(k * 16, 16)] = jnp.ones((16,), jnp.float32)
    for k in range(RPT // 16):
        zb[pl.ds(k * 16, 16)] = jnp.zeros((16,), jnp.float32)
    pltpu.sync_copy(zb, acc.at[pl.ds(s * RPT, RPT)])
    plsc.subcore_barrier()
    pltpu.sync_copy(dst_hbm.at[tid], didx)

    def _count(j, carry):
        pltpu.sync_copy(ones_v, acc.at[didx.at[j]], add=True)
        return carry

    lax.fori_loop(0, NCHUNK, _count, 0)
    plsc.subcore_barrier()
    pltpu.sync_copy(acc.at[pl.ds(s * RPT, RPT)], out_hbm.at[c, pl.ds(s * RPT, RPT)])


@functools.cache
def _deg_call():
    return pl.kernel(
        _deg_body,
        out_type=jax.ShapeDtypeStruct((2, NPAD), jnp.float32),
        mesh=plsc.VectorSubcoreMesh(core_axis_name="c", subcore_axis_name="s"),
        scratch_types=[
            pltpu.VMEM((NCHUNK, CHUNK), jnp.int32),
            pltpu.VMEM((CHUNK,), jnp.float32),
            pltpu.VMEM((RPT,), jnp.float32),
            pltpu.VMEM_SHARED((NPAD,), jnp.float32),
        ],
    )


def _edge_body(src_hbm, dst_hbm, gp_hbm, out_hbm, sidx, didx,
               rows0, rows1, rows2, rows3,
               acc, sg0, sg1, sg2, sg3, ss0, ss1, ss2, ss3):
    c = lax.axis_index("c")
    s = lax.axis_index("s")
    tid = c * TPV + s
    rows = (rows0, rows1, rows2, rows3)
    sg = (sg0, sg1, sg2, sg3)
    ss = (ss0, ss1, ss2, ss3)

    def _zrow(r, carry):
        for k in range(D // 16):
            rows0[r, pl.ds(k * 16, 16)] = jnp.zeros((16,), jnp.float32)
        return carry

    lax.fori_loop(0, ECH, _zrow, 0)
    for k in range(RPT // ECH):
        pltpu.sync_copy(rows0, acc.at[pl.ds(s * RPT + k * ECH, ECH)])
    plsc.subcore_barrier()

    def _chunk(m, carry):
        j = NBUF * m
        for b in range(NBUF):
            pltpu.make_async_copy(gp_hbm.at[sidx.at[j + b]], rows[b], sg[b]).wait()
            pltpu.async_copy(rows[b], acc.at[didx.at[j + b]], ss[b], add=True)
        for b in range(NBUF):
            pltpu.make_async_copy(rows[b], acc.at[didx.at[j + b]], ss[b]).wait()

            @pl.when(j + b + NBUF < EQ)
            def _():
                pltpu.async_copy(gp_hbm.at[sidx.at[j + b + NBUF]], rows[b], sg[b])

        return carry

    for q in range(ENCH // EQ):
        pltpu.sync_copy(src_hbm.at[tid, pl.ds(q * EQ, EQ)], sidx)
        pltpu.sync_copy(dst_hbm.at[tid, pl.ds(q * EQ, EQ)], didx)
        for b in range(NBUF):
            pltpu.async_copy(gp_hbm.at[sidx.at[b]], rows[b], sg[b])
        lax.fori_loop(0, EQ // NBUF, _chunk, 0)
    plsc.subcore_barrier()
    pltpu.sync_copy(acc.at[pl.ds(s * RPT, RPT)], out_hbm.at[c, pl.ds(s * RPT, RPT)])


@functools.cache
def _edge_call():
    return pl.kernel(
        _edge_body,
        out_type=jax.ShapeDtypeStruct((2, NPAD, D), jnp.float32),
        mesh=plsc.VectorSubcoreMesh(core_axis_name="c", subcore_axis_name="s"),
        scratch_types=[
            pltpu.VMEM((EQ, ECH), jnp.int32),
            pltpu.VMEM((EQ, ECH), jnp.int32),
            pltpu.VMEM((ECH, D), jnp.float32),
            pltpu.VMEM((ECH, D), jnp.float32),
            pltpu.VMEM((ECH, D), jnp.float32),
            pltpu.VMEM((ECH, D), jnp.float32),
            pltpu.VMEM_SHARED((NPAD, D), jnp.float32),
        ] + [pltpu.SemaphoreType.DMA] * 8,
    )


def _l1_body(x_ref, deg_ref, w_ref, gp_ref, dinv_ref):
    dinv = lax.rsqrt(deg_ref[0] + 1.0)
    dinv_ref[0] = dinv
    g = jnp.dot(x_ref[0], w_ref[...], preferred_element_type=jnp.float32)
    gp_ref[0] = g * dinv


def _l2_body(t_ref, gp_ref, dinv_ref, b_ref, w_ref, u_ref, gpn_ref):
    dinv = dinv_ref[0]
    u = jnp.maximum((t_ref[0] + gp_ref[0]) * dinv + b_ref[...], 0.0)
    u_ref[0] = u
    gpn_ref[0] = jnp.dot(u, w_ref[...], preferred_element_type=jnp.float32) * dinv


def _l3_body(t_ref, gp_ref, up_ref, dinv_ref, b_ref, w_ref, u_ref, gpn_ref):
    dinv = dinv_ref[0]
    u = jnp.maximum((t_ref[0] + gp_ref[0]) * dinv + b_ref[...], 0.0) + up_ref[0]
    u_ref[0] = u
    gpn_ref[0] = jnp.dot(u, w_ref[...], preferred_element_type=jnp.float32) * dinv


def _pool_body(t_ref, gp_ref, up_ref, dinv_ref, b_ref, batch_ref,
               sums_ref, cnt_ref):
    i = pl.program_id(1)
    u4 = jnp.maximum((t_ref[0] + gp_ref[0]) * dinv_ref[0] + b_ref[...], 0.0) \
        + up_ref[0]
    bvec = batch_ref[0]
    rows = lax.broadcasted_iota(jnp.int32, (B, RB), 0)
    oh = (bvec == rows).astype(jnp.float32)
    psum = jnp.dot(oh, u4, preferred_element_type=jnp.float32,
                   precision=lax.Precision.HIGHEST)
    pcnt = jnp.sum(oh, axis=1, keepdims=True)

    @pl.when(i == 0)
    def _():
        sums_ref[0] = psum
        cnt_ref[0] = pcnt

    @pl.when(i > 0)
    def _():
        sums_ref[0] += psum
        cnt_ref[0] += pcnt


def _head_body(sums_ref, cnt_ref, pw1, pb1, g1, be1, pw2, pb2, g2, be2,
               qw1, qb1, qw2, qb2, p_ref, z_ref):
    h = sums_ref[0] / jnp.maximum(cnt_ref[0], 1.0)
    z = jnp.dot(h, pw1[...], preferred_element_type=jnp.float32) + pb1[...]
    mu = jnp.mean(z, axis=0, keepdims=True)
    var = jnp.mean((z - mu) ** 2, axis=0, keepdims=True)
    z = jnp.maximum(g1[...] * (z - mu) * lax.rsqrt(var + 1e-5) + be1[...], 0.0)
    z = jnp.dot(z, pw2[...], preferred_element_type=jnp.float32) + pb2[...]
    mu = jnp.mean(z, axis=0, keepdims=True)
    var = jnp.mean((z - mu) ** 2, axis=0, keepdims=True)
    z = g2[...] * (z - mu) * lax.rsqrt(var + 1e-5) + be2[...]
    z_ref[0] = z
    q = jnp.maximum(
        jnp.dot(z, qw1[...], preferred_element_type=jnp.float32) + qb1[...], 0.0)
    p_ref[0] = jnp.dot(q, qw2[...], preferred_element_type=jnp.float32) + qb2[...]


def _row_spec(last=D):
    return pl.BlockSpec((1, RB, last), lambda v, i: (v, i, 0))


def _full_spec(shape):
    return pl.BlockSpec(shape, lambda v, i: tuple(0 for _ in shape))


def _tc_layer1(x_s, deg, w1):
    return pl.pallas_call(
        _l1_body,
        grid=(2, GI),
        in_specs=[_row_spec(), _row_spec(1), _full_spec((D, D))],
        out_specs=[_row_spec(), _row_spec(1)],
        out_shape=[jax.ShapeDtypeStruct((2, NPAD, D), jnp.float32),
                   jax.ShapeDtypeStruct((2, NPAD, 1), jnp.float32)],
    )(x_s, deg, w1)


def _tc_layer2(t, gp, dinv, b, w):
    return pl.pallas_call(
        _l2_body,
        grid=(2, GI),
        in_specs=[_row_spec(), _row_spec(), _row_spec(1),
                  _full_spec((1, D)), _full_spec((D, D))],
        out_specs=[_row_spec(), _row_spec()],
        out_shape=[jax.ShapeDtypeStruct((2, NPAD, D), jnp.float32),
                   jax.ShapeDtypeStruct((2, NPAD, D), jnp.float32)],
    )(t, gp, dinv, b, w)


def _tc_layer3(t, gp, up, dinv, b, w):
    return pl.pallas_call(
        _l3_body,
        grid=(2, GI),
        in_specs=[_row_spec(), _row_spec(), _row_spec(), _row_spec(1),
                  _full_spec((1, D)), _full_spec((D, D))],
        out_specs=[_row_spec(), _row_spec()],
        out_shape=[jax.ShapeDtypeStruct((2, NPAD, D), jnp.float32),
                   jax.ShapeDtypeStruct((2, NPAD, D), jnp.float32)],
    )(t, gp, up, dinv, b, w)


def _tc_pool(t, gp, up, dinv, b, batch_s):
    return pl.pallas_call(
        _pool_body,
        grid=(2, GI),
        in_specs=[_row_spec(), _row_spec(), _row_spec(), _row_spec(1),
                  _full_spec((1, D)),
                  pl.BlockSpec((1, 1, RB), lambda v, i: (v, 0, i))],
        out_specs=[pl.BlockSpec((1, B, D), lambda v, i: (v, 0, 0)),
                   pl.BlockSpec((1, B, 1), lambda v, i: (v, 0, 0))],
        out_shape=[jax.ShapeDtypeStruct((2, B, D), jnp.float32),
                   jax.ShapeDtypeStruct((2, B, 1), jnp.float32)],
    )(t, gp, up, dinv, b, batch_s)


def _tc_head(sums, cnt, p):
    specs = [pl.BlockSpec((1, B, D), lambda v: (v, 0, 0)),
             pl.BlockSpec((1, B, 1), lambda v: (v, 0, 0))]
    args = [sums, cnt]
    for w, bias in ((p['pW1'], p['pb1']), (p['g1'], p['be1']),
                    (p['pW2'], p['pb2']), (p['g2'], p['be2']),
                    (p['qW1'], p['qb1']), (p['qW2'], p['qb2'])):
        for a in (w, bias):
            a2 = a.reshape((1, -1)) if a.ndim == 1 else a
            args.append(a2)
            specs.append(pl.BlockSpec(a2.shape, lambda v: (0, 0)))
    return pl.pallas_call(
        _head_body,
        grid=(2,),
        in_specs=specs,
        out_specs=[pl.BlockSpec((1, B, D), lambda v: (v, 0, 0)),
                   pl.BlockSpec((1, B, D), lambda v: (v, 0, 0))],
        out_shape=[jax.ShapeDtypeStruct((2, B, D), jnp.float32),
                   jax.ShapeDtypeStruct((2, B, D), jnp.float32)],
    )(*args)


def _prep_edges(ei, view):
    src = ei[0].astype(jnp.int32)
    dst = ei[1].astype(jnp.int32)
    pad = EPV - E
    ar = jnp.arange(pad, dtype=jnp.int32)
    src = jnp.concatenate([src, ar % N]) + view * NPAD
    dst = jnp.concatenate([dst, N + (ar % (NPAD - N))])
    return src, dst


def kernel(x1, edge_index1, batch1, x2, edge_index2, batch2, params):
    p = params
    x_s = jnp.pad(jnp.stack([x1, x2]).astype(jnp.float32),
                  ((0, 0), (0, NPAD - N), (0, 0)))
    s1, d1 = _prep_edges(edge_index1, 0)
    s2, d2 = _prep_edges(edge_index2, 1)
    e_src = jnp.stack([s1, s2]).reshape(2 * TPV, ENCH, ECH)
    e_dst = jnp.stack([d1, d2]).reshape(2 * TPV, NCHUNK, CHUNK)
    e_dst64 = e_dst.reshape(2 * TPV, ENCH, ECH)
    batch_s = jnp.pad(jnp.stack([batch1, batch2]).astype(jnp.int32),
                      ((0, 0), (0, NPAD - N)),
                      constant_values=B).reshape(2, 1, NPAD)

    deg = _deg_call()(e_dst).reshape(2, NPAD, 1)
    gp1, dinv = _tc_layer1(x_s, deg, p['W1'])
    t1 = _edge_call()(e_src, e_dst64, gp1.reshape(2 * NPAD, D))
    u2, gp2 = _tc_layer2(t1, gp1, dinv, p['b1'].reshape(1, D), p['W2'])
    t2 = _edge_call()(e_src, e_dst64, gp2.reshape(2 * NPAD, D))
    u3, gp3 = _tc_layer3(t2, gp2, u2, dinv, p['b2'].reshape(1, D), p['W3'])
    t3 = _edge_call()(e_src, e_dst64, gp3.reshape(2 * NPAD, D))
    sums, cnt = _tc_pool(t3, gp3, u3, dinv, p['b3'].reshape(1, D), batch_s)
    p_out, z_out = _tc_head(sums, cnt, p)
    return (p_out[0], p_out[1], z_out[0], z_out[1])
